# baseline (speedup 1.0000x reference)
import functools

import jax
import jax.numpy as jnp
from jax import lax
from jax.experimental import pallas as pl
from jax.experimental.pallas import tpu as pltpu
from jax.experimental.pallas import tpu_sc as plsc

NC, NS, L = 2, 16, 16
NW = NC * NS
B, SEQ = 4096, 200
WD, PD, OD = 64, 16, 96
BW = B // NW
CT, BT = OD // 8, B // 128
NG = BW // L


@functools.partial(
    pl.kernel,
    out_type=jax.ShapeDtypeStruct((SEQ, CT, BT, 8, 128), jnp.float32),
    mesh=plsc.VectorSubcoreMesh(core_axis_name="c", subcore_axis_name="s"),
    scratch_types=[
        pltpu.VMEM((2, BW), jnp.int32),
        pltpu.VMEM((2, BW), jnp.int32),
        pltpu.VMEM((2, BW), jnp.int32),
        pltpu.VMEM((2, BW, WD), jnp.float32),
        pltpu.VMEM((2, CT, 8, 128), jnp.float32),
        pltpu.VMEM((512 * PD,), jnp.float32),
        pltpu.VMEM((512 * PD,), jnp.float32),
        pltpu.SemaphoreType.DMA,
        pltpu.SemaphoreType.DMA,
        pltpu.SemaphoreType.DMA,
        pltpu.SemaphoreType.DMA,
    ],
    compiler_params=pltpu.CompilerParams(use_tc_tiling_on_sc=False,
                                         needs_layout_passes=False),
)
def _embed_sc(words_hbm, head_hbm, tail_hbm, wt_hbm, ht_hbm, tt_hbm,
              out_hbm, widx_v, hidx_v, tidx_v, wrow_v, stg_v, ht_v, tt_v,
              gsem0, gsem1, wsem0, wsem1):
    wid = lax.axis_index("s") * NC + lax.axis_index("c")
    b0 = wid * BW

    pltpu.sync_copy(ht_hbm, ht_v)
    pltpu.sync_copy(tt_hbm, tt_v)

    gsems = (gsem0, gsem1)
    wsems = (wsem0, wsem1)

    def load_and_fire(l, buf):
        pltpu.sync_copy(words_hbm.at[pl.ds(l * B + b0, BW)], widx_v.at[buf])
        pltpu.sync_copy(head_hbm.at[pl.ds(l * B + b0, BW)], hidx_v.at[buf])
        pltpu.sync_copy(tail_hbm.at[pl.ds(l * B + b0, BW)], tidx_v.at[buf])
        pltpu.async_copy(wt_hbm.at[widx_v.at[buf]], wrow_v.at[buf], gsems[buf])

    load_and_fire(0, 0)

    def substep(l, cur):
        nxt = 1 - cur

        @pl.when(l + 1 < SEQ)
        def _():
            load_and_fire(l + 1, nxt)

        pltpu.make_async_copy(
            wt_hbm.at[widx_v.at[cur]], wrow_v.at[cur], gsems[cur]).wait()
        @pl.when(l >= 2)
        def _():
            pltpu.make_async_copy(
                stg_v.at[cur], out_hbm.at[l - 2, :, wid], wsems[cur]).wait()

        def group_body(g, _):
            sl = pl.ds(g * L, L)
            widx16 = widx_v[cur, sl]
            hoff = hidx_v[cur, sl] * PD
            toff = tidx_v[cur, sl] * PD
            toks = g * L + lax.iota(jnp.int32, L)
            mpad = widx16 == 0
            for c in range(WD):
                v = plsc.load_gather(
                    wrow_v, [jnp.full((L,), cur, jnp.int32), toks,
                             jnp.full((L,), c, jnp.int32)])
                v = jnp.where(mpad, 0.0, v)
                stg_v[cur, c // 8, c % 8, sl] = v
            for c in range(PD):
                hv = plsc.load_gather(ht_v, [hoff + c])
                stg_v[cur, (WD + c) // 8, (WD + c) % 8, sl] = hv
                tv = plsc.load_gather(tt_v, [toff + c])
                stg_v[cur, (WD + PD + c) // 8, (WD + PD + c) % 8, sl] = tv
            return 0

        lax.fori_loop(0, NG, group_body, 0)

        pltpu.async_copy(stg_v.at[cur], out_hbm.at[l, :, wid], wsems[cur])

    def pair_body(p, _):
        substep(2 * p, 0)
        substep(2 * p + 1, 1)
        return 0

    lax.fori_loop(0, SEQ // 2, pair_body, 0)

    pltpu.make_async_copy(
        stg_v.at[0], out_hbm.at[SEQ - 2, :, wid], wsem0).wait()
    pltpu.make_async_copy(
        stg_v.at[1], out_hbm.at[SEQ - 1, :, wid], wsem1).wait()


def kernel(words, head_pos, tail_pos, word_table, head_pos_table, tail_pos_table):
    ht = head_pos_table.at[0].set(0.0).reshape(512 * PD)
    tt = tail_pos_table.at[0].set(0.0).reshape(512 * PD)
    wT = words.T.reshape(SEQ * B)
    hT = head_pos.T.reshape(SEQ * B)
    tT = tail_pos.T.reshape(SEQ * B)
    out5 = _embed_sc(wT, hT, tT, word_table, ht, tt)
    return out5.transpose(2, 4, 0, 1, 3).reshape(B, SEQ, OD)


# speedup vs baseline: 1.4760x; 1.4760x over previous
import functools

import jax
import jax.numpy as jnp
from jax import lax
from jax.experimental import pallas as pl
from jax.experimental.pallas import tpu as pltpu
from jax.experimental.pallas import tpu_sc as plsc

NC, NS, L = 2, 16, 16
NW = NC * NS
B, SEQ = 4096, 200
WD, PD, OD = 64, 16, 96
BW = B // NW
CT, BT = OD // 8, B // 128
WCT = WD // 8
PCT = CT - WCT
NG = BW // L
PITCH = BW + 1


@functools.partial(
    pl.kernel,
    out_type=jax.ShapeDtypeStruct((SEQ, CT, BT, 8, 128), jnp.float32),
    mesh=plsc.VectorSubcoreMesh(core_axis_name="c", subcore_axis_name="s"),
    scratch_types=[
        pltpu.VMEM((2, BW), jnp.int32),
        pltpu.VMEM((2, BW), jnp.int32),
        pltpu.VMEM((2, BW), jnp.int32),
        pltpu.VMEM((2, BW, WD), jnp.float32),
        pltpu.VMEM((2, WCT, 8, PITCH), jnp.float32),
        pltpu.VMEM((2, PCT, 8, 128), jnp.float32),
        pltpu.VMEM((512 * PD,), jnp.float32),
        pltpu.VMEM((512 * PD,), jnp.float32),
        pltpu.SemaphoreType.DMA,
        pltpu.SemaphoreType.DMA,
        pltpu.SemaphoreType.DMA,
        pltpu.SemaphoreType.DMA,
        pltpu.SemaphoreType.DMA,
        pltpu.SemaphoreType.DMA,
    ],
    compiler_params=pltpu.CompilerParams(use_tc_tiling_on_sc=False,
                                         needs_layout_passes=False),
)
def _embed_sc(words_hbm, head_hbm, tail_hbm, wt_hbm, ht_hbm, tt_hbm,
              out_hbm, widx_v, hidx_v, tidx_v, wrow_v, stgw_v, stgp_v,
              ht_v, tt_v, gsem0, gsem1, wsemw0, wsemw1, wsemp0, wsemp1):
    wid = lax.axis_index("s") * NC + lax.axis_index("c")
    b0 = wid * BW

    pltpu.sync_copy(ht_hbm, ht_v)
    pltpu.sync_copy(tt_hbm, tt_v)

    gsems = (gsem0, gsem1)
    wsemws = (wsemw0, wsemw1)
    wsemps = (wsemp0, wsemp1)
    iota = lax.iota(jnp.int32, L)
    ctv = [(jnp.full((L,), cb * L, jnp.int32) + iota) // 8 for cb in range(4)]
    civ = [(jnp.full((L,), cb * L, jnp.int32) + iota) % 8 for cb in range(4)]
    zf = jnp.zeros((L,), jnp.float32)

    def load_and_fire(l, buf):
        pltpu.sync_copy(words_hbm.at[pl.ds(l * B + b0, BW)], widx_v.at[buf])
        pltpu.sync_copy(head_hbm.at[pl.ds(l * B + b0, BW)], hidx_v.at[buf])
        pltpu.sync_copy(tail_hbm.at[pl.ds(l * B + b0, BW)], tidx_v.at[buf])
        pltpu.async_copy(wt_hbm.at[widx_v.at[buf]], wrow_v.at[buf], gsems[buf])

    load_and_fire(0, 0)

    def substep(l, cur):
        nxt = 1 - cur

        @pl.when(l + 1 < SEQ)
        def _():
            load_and_fire(l + 1, nxt)

        pltpu.make_async_copy(
            wt_hbm.at[widx_v.at[cur]], wrow_v.at[cur], gsems[cur]).wait()
        @pl.when(l >= 2)
        def _():
            pltpu.make_async_copy(
                stgw_v.at[cur, :, :, pl.ds(0, BW)],
                out_hbm.at[l - 2, pl.ds(0, WCT), wid], wsemws[cur]).wait()
            pltpu.make_async_copy(
                stgp_v.at[cur],
                out_hbm.at[l - 2, pl.ds(WCT, PCT), wid], wsemps[cur]).wait()

        curv = jnp.full((L,), cur, jnp.int32)

        def group_body(g, _):
            sl = pl.ds(g * L, L)
            toks = g * L + iota
            hoff = hidx_v[cur, sl] * PD
            toff = tidx_v[cur, sl] * PD
            for c in range(PD):
                stgp_v[cur, c // 8, c % 8, sl] = plsc.load_gather(ht_v, [hoff + c])
            for c in range(PD):
                c2 = PD + c
                stgp_v[cur, c2 // 8, c2 % 8, sl] = plsc.load_gather(tt_v, [toff + c])

            for t in range(L):
                tok = g * L + t
                tokv = jnp.full((L,), tok, jnp.int32)
                for cb in range(4):
                    v = wrow_v[cur, tok, pl.ds(cb * L, L)]
                    plsc.store_scatter(stgw_v, [curv, ctv[cb], civ[cb], tokv], v)

            widx16 = widx_v[cur, sl]

            @pl.when(jnp.min(widx16) == 0)
            def _():
                msk = widx16 == 0
                for c in range(WD):
                    plsc.store_scatter(
                        stgw_v,
                        [curv, jnp.full((L,), c // 8, jnp.int32),
                         jnp.full((L,), c % 8, jnp.int32), toks],
                        zf, mask=msk)
            return 0

        lax.fori_loop(0, NG, group_body, 0)

        pltpu.async_copy(stgw_v.at[cur, :, :, pl.ds(0, BW)],
                         out_hbm.at[l, pl.ds(0, WCT), wid], wsemws[cur])
        pltpu.async_copy(stgp_v.at[cur],
                         out_hbm.at[l, pl.ds(WCT, PCT), wid], wsemps[cur])

    def pair_body(p, _):
        substep(2 * p, 0)
        substep(2 * p + 1, 1)
        return 0

    lax.fori_loop(0, SEQ // 2, pair_body, 0)

    for l, buf in ((SEQ - 2, 0), (SEQ - 1, 1)):
        pltpu.make_async_copy(
            stgw_v.at[buf, :, :, pl.ds(0, BW)],
            out_hbm.at[l, pl.ds(0, WCT), wid], wsemws[buf]).wait()
        pltpu.make_async_copy(
            stgp_v.at[buf],
            out_hbm.at[l, pl.ds(WCT, PCT), wid], wsemps[buf]).wait()


def kernel(words, head_pos, tail_pos, word_table, head_pos_table, tail_pos_table):
    ht = head_pos_table.at[0].set(0.0).reshape(512 * PD)
    tt = tail_pos_table.at[0].set(0.0).reshape(512 * PD)
    wT = words.T.reshape(SEQ * B)
    hT = head_pos.T.reshape(SEQ * B)
    tT = tail_pos.T.reshape(SEQ * B)
    out5 = _embed_sc(wT, hT, tT, word_table, ht, tt)
    return out5.transpose(2, 4, 0, 1, 3).reshape(B, SEQ, OD)


# speedup vs baseline: 1.9723x; 1.3363x over previous
import functools

import jax
import jax.numpy as jnp
from jax import lax
from jax.experimental import pallas as pl
from jax.experimental.pallas import tpu as pltpu
from jax.experimental.pallas import tpu_sc as plsc

NC, NS, L = 2, 16, 16
NW = NC * NS
B, SEQ = 4096, 200
WD, PD, OD = 64, 16, 96
LT = SEQ // 8
B_PER_W = B // NW
NSEQ = 4
NCHUNK = B_PER_W // NSEQ
SPLITS = ((0, 96), (96, 104))


@functools.partial(
    pl.kernel,
    out_type=jax.ShapeDtypeStruct((B, SEQ, 128), jnp.float32),
    mesh=plsc.VectorSubcoreMesh(core_axis_name="c", subcore_axis_name="s"),
    scratch_types=[
        pltpu.VMEM((NSEQ, SEQ), jnp.int32),
        pltpu.VMEM((NSEQ, SEQ), jnp.int32),
        pltpu.VMEM((NSEQ, SEQ), jnp.int32),
        pltpu.VMEM((NSEQ, SEQ, WD), jnp.float32),
        pltpu.VMEM((NSEQ, SEQ, PD), jnp.float32),
        pltpu.VMEM((NSEQ, SEQ, PD), jnp.float32),
        pltpu.SemaphoreType.DMA,
    ],
    compiler_params=pltpu.CompilerParams(use_tc_tiling_on_sc=False,
                                         needs_layout_passes=False),
)
def _embed_sc(words_hbm, head_hbm, tail_hbm, wt_hbm, ht_hbm, tt_hbm,
              out_hbm, widx_v, hidx_v, tidx_v, wrow_v, hrow_v, trow_v, sem):
    wid = lax.axis_index("s") * NC + lax.axis_index("c")
    seq0 = wid * B_PER_W

    def chunk_body(ci, _):
        b0 = seq0 + ci * NSEQ

        pltpu.sync_copy(words_hbm.at[pl.ds(b0, NSEQ)], widx_v)
        pltpu.sync_copy(head_hbm.at[pl.ds(b0, NSEQ)], hidx_v)
        pltpu.sync_copy(tail_hbm.at[pl.ds(b0, NSEQ)], tidx_v)

        copies = []
        for i in range(NSEQ):
            for off, width in SPLITS:
                isl = pl.ds(off, width)
                copies.append(pltpu.async_copy(
                    wt_hbm.at[widx_v.at[i, isl]], wrow_v.at[i, isl], sem))
                copies.append(pltpu.async_copy(
                    ht_hbm.at[hidx_v.at[i, isl]], hrow_v.at[i, isl], sem))
                copies.append(pltpu.async_copy(
                    tt_hbm.at[tidx_v.at[i, isl]], trow_v.at[i, isl], sem))
        for c in copies:
            c.wait()

        def fixup_body(i, _):
            for o in list(range(0, SEQ - L, L)) + [SEQ - L]:
                idxs = widx_v[i, pl.ds(o, L)]
                msk = idxs == 0

                @pl.when(jnp.min(idxs) == 0)
                def _():
                    toks = o + lax.iota(jnp.int32, L)
                    seqv = jnp.full((L,), i, jnp.int32)
                    zf = jnp.zeros((L,), jnp.float32)
                    for col in range(WD):
                        plsc.store_scatter(
                            wrow_v,
                            [seqv, toks, jnp.full((L,), col, jnp.int32)],
                            zf, mask=msk)
            return 0

        lax.fori_loop(0, NSEQ, fixup_body, 0)

        dst = out_hbm.at[pl.ds(b0, NSEQ)]
        pltpu.sync_copy(wrow_v, dst.at[:, :, pl.ds(0, WD)])
        pltpu.sync_copy(hrow_v, dst.at[:, :, pl.ds(WD, PD)])
        pltpu.sync_copy(trow_v, dst.at[:, :, pl.ds(WD + PD, PD)])
        return 0

    lax.fori_loop(0, NCHUNK, chunk_body, 0)


def kernel(words, head_pos, tail_pos, word_table, head_pos_table, tail_pos_table):
    ht = head_pos_table.at[0].set(0.0)
    tt = tail_pos_table.at[0].set(0.0)
    img = _embed_sc(words, head_pos, tail_pos, word_table, ht, tt)
    return img[:, :, :OD]


# speedup vs baseline: 2.0581x; 1.0435x over previous
import functools

import jax
import jax.numpy as jnp
from jax import lax
from jax.experimental import pallas as pl
from jax.experimental.pallas import tpu as pltpu
from jax.experimental.pallas import tpu_sc as plsc

NC, NS, L = 2, 16, 16
NW = NC * NS
B, SEQ = 4096, 200
WD, PD, OD = 64, 16, 96
B_PER_W = B // NW
NSEQ = 2
NCHUNK = B_PER_W // NSEQ
SPLITS = ((0, 96), (96, 104))


@functools.partial(
    pl.kernel,
    out_type=jax.ShapeDtypeStruct((B, SEQ, 128), jnp.float32),
    mesh=plsc.VectorSubcoreMesh(core_axis_name="c", subcore_axis_name="s"),
    scratch_types=[
        pltpu.VMEM((2, NSEQ, SEQ), jnp.int32),
        pltpu.VMEM((2, NSEQ, SEQ), jnp.int32),
        pltpu.VMEM((2, NSEQ, SEQ), jnp.int32),
        pltpu.VMEM((2, NSEQ, SEQ, WD), jnp.float32),
        pltpu.VMEM((2, NSEQ, SEQ, PD), jnp.float32),
        pltpu.VMEM((2, NSEQ, SEQ, PD), jnp.float32),
        pltpu.SemaphoreType.DMA,
        pltpu.SemaphoreType.DMA,
        pltpu.SemaphoreType.DMA,
        pltpu.SemaphoreType.DMA,
    ],
    compiler_params=pltpu.CompilerParams(use_tc_tiling_on_sc=False,
                                         needs_layout_passes=False),
)
def _embed_sc(words_hbm, head_hbm, tail_hbm, wt_hbm, ht_hbm, tt_hbm,
              out_hbm, widx_v, hidx_v, tidx_v, wrow_v, hrow_v, trow_v,
              gsem0, gsem1, wsem0, wsem1):
    wid = lax.axis_index("s") * NC + lax.axis_index("c")
    seq0 = wid * B_PER_W
    gsems = (gsem0, gsem1)
    wsems = (wsem0, wsem1)

    def gather_args(buf):
        args = []
        for i in range(NSEQ):
            for off, width in SPLITS:
                isl = pl.ds(off, width)
                args.append((wt_hbm.at[widx_v.at[buf, i, isl]],
                             wrow_v.at[buf, i, isl]))
                args.append((ht_hbm.at[hidx_v.at[buf, i, isl]],
                             hrow_v.at[buf, i, isl]))
                args.append((tt_hbm.at[tidx_v.at[buf, i, isl]],
                             trow_v.at[buf, i, isl]))
        return args

    def write_args(ci, buf):
        dst = out_hbm.at[pl.ds(seq0 + ci * NSEQ, NSEQ)]
        return ((wrow_v.at[buf], dst.at[:, :, pl.ds(0, WD)]),
                (hrow_v.at[buf], dst.at[:, :, pl.ds(WD, PD)]),
                (trow_v.at[buf], dst.at[:, :, pl.ds(WD + PD, PD)]))

    def load_and_fire(ci, buf):
        b0 = seq0 + ci * NSEQ
        pltpu.sync_copy(words_hbm.at[pl.ds(b0, NSEQ)], widx_v.at[buf])
        pltpu.sync_copy(head_hbm.at[pl.ds(b0, NSEQ)], hidx_v.at[buf])
        pltpu.sync_copy(tail_hbm.at[pl.ds(b0, NSEQ)], tidx_v.at[buf])
        for src, dstv in gather_args(buf):
            pltpu.async_copy(src, dstv, gsems[buf])

    load_and_fire(0, 0)

    def substep(ci, cur):
        nxt = 1 - cur

        @pl.when(ci + 1 < NCHUNK)
        def _():
            load_and_fire(ci + 1, nxt)

        for src, dstv in gather_args(cur):
            pltpu.make_async_copy(src, dstv, gsems[cur]).wait()

        @pl.when(ci >= 2)
        def _():
            for src, dstv in write_args(ci - 2, cur):
                pltpu.make_async_copy(src, dstv, wsems[cur]).wait()

        def fixup_body(i, _):
            for o in list(range(0, SEQ - L, L)) + [SEQ - L]:
                idxs = widx_v[cur, i, pl.ds(o, L)]
                msk = idxs == 0

                @pl.when(jnp.min(idxs) == 0)
                def _():
                    toks = o + lax.iota(jnp.int32, L)
                    bufv = jnp.full((L,), cur, jnp.int32)
                    seqv = jnp.full((L,), i, jnp.int32)
                    zf = jnp.zeros((L,), jnp.float32)
                    for col in range(WD):
                        plsc.store_scatter(
                            wrow_v,
                            [bufv, seqv, toks, jnp.full((L,), col, jnp.int32)],
                            zf, mask=msk)
            return 0

        lax.fori_loop(0, NSEQ, fixup_body, 0)

        for src, dstv in write_args(ci, cur):
            pltpu.async_copy(src, dstv, wsems[cur])

    def pair_body(p, _):
        substep(2 * p, 0)
        substep(2 * p + 1, 1)
        return 0

    lax.fori_loop(0, NCHUNK // 2, pair_body, 0)

    for ci, buf in ((NCHUNK - 2, 0), (NCHUNK - 1, 1)):
        for src, dstv in write_args(ci, buf):
            pltpu.make_async_copy(src, dstv, wsems[buf]).wait()


def kernel(words, head_pos, tail_pos, word_table, head_pos_table, tail_pos_table):
    ht = head_pos_table.at[0].set(0.0)
    tt = tail_pos_table.at[0].set(0.0)
    img = _embed_sc(words, head_pos, tail_pos, word_table, ht, tt)
    return img[:, :, :OD]
